# baseline (speedup 1.0000x reference)
import functools

import jax
import jax.numpy as jnp
from jax import lax
from jax.experimental import pallas as pl
from jax.experimental.pallas import tpu as pltpu
from jax.experimental.pallas import tpu_sc as plsc

M = 100000
D = 64
B = 16384
M2 = M // 2
D2 = 2 * D
NC, NS = 2, 16
NW = NC * NS
BPW = B // NW
BPT = B // NS
NBLK = 6
BLK = 8336
SCRAP = 512
CHUNK = 528
DCH = 128

_mesh = plsc.VectorSubcoreMesh(core_axis_name="c", subcore_axis_name="s")


@functools.partial(
    pl.kernel, mesh=_mesh,
    out_type=jax.ShapeDtypeStruct((B, D2), jnp.float32),
    scratch_types=[
        pltpu.VMEM((BPW // 128, 128), jnp.int32),
        pltpu.VMEM((BPW, D2), jnp.float32),
    ],
)
def _sc_gather(mem2_hbm, idx_hbm, old2_hbm, idx_v, rows_v):
    wid = lax.axis_index("s") * NC + lax.axis_index("c")
    pltpu.sync_copy(idx_hbm.at[wid], idx_v)
    for j in range(BPW // 128):
        pltpu.sync_copy(mem2_hbm.at[idx_v.at[j]],
                        rows_v.at[pl.ds(j * 128, 128)])
    pltpu.sync_copy(rows_v, old2_hbm.at[pl.ds(wid * BPW, BPW)])


@functools.partial(
    pl.kernel, mesh=_mesh,
    out_type=jax.ShapeDtypeStruct((M2, D2), jnp.float32),
    scratch_types=[
        pltpu.VMEM_SHARED((BLK + SCRAP, D2), jnp.float32),
        pltpu.VMEM((BPT // 128, 128), jnp.int32),
        pltpu.VMEM((BPT // 128, 128), jnp.int32),
        pltpu.VMEM((DCH, D2), jnp.float32),
    ],
)
def _sc_scatter(mem2_hbm, delta2_hbm, idx_hbm, out2_hbm,
                blk_sh, idx_v, lidx_v, delta_v):
    cid = lax.axis_index("c")
    sid = lax.axis_index("s")
    pltpu.sync_copy(idx_hbm.at[sid], idx_v)
    cs = jnp.minimum(sid * CHUNK, BLK - CHUNK)
    for b in range(NBLK // NC):
        blk = cid * (NBLK // NC) + b
        start = jnp.minimum(blk * BLK, M2 - BLK)
        pltpu.sync_copy(mem2_hbm.at[pl.ds(start + cs, CHUNK)],
                        blk_sh.at[pl.ds(cs, CHUNK)])
        for r in range(BPT // 128):
            for k in range(128 // 16):
                iv = idx_v[r, pl.ds(k * 16, 16)]
                inb = (iv >= start) & (iv < start + BLK)
                loc = iv - start
                scrap = BLK + (iv & (SCRAP - 1))
                lidx_v[r, pl.ds(k * 16, 16)] = jnp.where(inb, loc, scrap)
        plsc.subcore_barrier()
        for p in range(BPT // DCH):
            pltpu.sync_copy(delta2_hbm.at[pl.ds(sid * BPT + p * DCH, DCH)],
                            delta_v)
            for r in range(DCH // 128):
                pltpu.sync_copy(delta_v.at[pl.ds(r * 128, 128)],
                                blk_sh.at[lidx_v.at[p * (DCH // 128) + r]],
                                add=True)
        plsc.subcore_barrier()
        pltpu.sync_copy(blk_sh.at[pl.ds(cs, CHUNK)],
                        out2_hbm.at[pl.ds(start + cs, CHUNK)])
        plsc.subcore_barrier()


def _delta_body(old2_ref, val_ref, par_ref, wg_ref, bg_ref, wv_ref, out_ref):
    p = par_ref[0]
    old2 = old2_ref[...]
    old = old2[:, :D] * (1.0 - p) + old2[:, D:] * p
    v = val_ref[...]
    pre = jnp.dot(old + v, wg_ref[...], preferred_element_type=jnp.float32)
    gate = jax.nn.sigmoid(pre + bg_ref[...])
    upd = jnp.tanh(jnp.dot(v, wv_ref[...], preferred_element_type=jnp.float32))
    d = gate * (upd - old)
    out_ref[...] = jnp.concatenate([d * (1.0 - p), d * p], axis=1)


_TCB = 2048


def _tc_delta(old2, values, par, W_gate, b_gate, W_val):
    return pl.pallas_call(
        _delta_body,
        grid=(B // _TCB,),
        in_specs=[
            pl.BlockSpec((_TCB, D2), lambda i: (i, 0)),
            pl.BlockSpec((_TCB, D), lambda i: (i, 0)),
            pl.BlockSpec((1, _TCB, 1), lambda i: (i, 0, 0)),
            pl.BlockSpec((D, D), lambda i: (0, 0)),
            pl.BlockSpec((1, D), lambda i: (0, 0)),
            pl.BlockSpec((D, D), lambda i: (0, 0)),
        ],
        out_specs=pl.BlockSpec((_TCB, D2), lambda i: (i, 0)),
        out_shape=jax.ShapeDtypeStruct((B, D2), jnp.float32),
    )(old2, values, par, W_gate, b_gate, W_val)


def kernel(memory, indices, values, W_gate, b_gate, W_val):
    idx = indices.astype(jnp.int32)
    idxp = idx >> 1
    par = (idx & 1).astype(jnp.float32).reshape(B // _TCB, _TCB, 1)
    mem2 = memory.reshape(M2, D2)
    old2 = _sc_gather(mem2, idxp.reshape(NW, BPW // 128, 128))
    delta2 = _tc_delta(old2, values, par, W_gate, b_gate.reshape(1, D), W_val)
    out2 = _sc_scatter(mem2, delta2, idxp.reshape(NS, BPT // 128, 128))
    return out2.reshape(M, D)


# speedup vs baseline: 1.0198x; 1.0198x over previous
import functools

import jax
import jax.numpy as jnp
from jax import lax
from jax.experimental import pallas as pl
from jax.experimental.pallas import tpu as pltpu
from jax.experimental.pallas import tpu_sc as plsc

M = 100000
D = 64
B = 16384
M2 = M // 2
D2 = 2 * D
HS = 1024
NC, NS = 2, 16
NW = NC * NS
BPW = B // NW
BPT = B // NS
NBLK = 6
BLK = 8336
SCRAP = 512
NCH = BPT // 128

_mesh = plsc.VectorSubcoreMesh(core_axis_name="c", subcore_axis_name="s")


@functools.partial(
    pl.kernel, mesh=_mesh,
    out_type=jax.ShapeDtypeStruct((B, D2), jnp.float32),
    scratch_types=[
        pltpu.VMEM((BPW // 128, 128), jnp.int32),
        pltpu.VMEM((BPW, D2), jnp.float32),
    ],
)
def _sc_gather(bank_ref, idx_hbm, old2_hbm, idx_v, rows_v):
    wid = lax.axis_index("s") * NC + lax.axis_index("c")
    pltpu.sync_copy(idx_hbm.at[wid], idx_v)
    for j in range(BPW // 128):
        pltpu.sync_copy(bank_ref.at[idx_v.at[j]],
                        rows_v.at[pl.ds(j * 128, 128)])
    pltpu.sync_copy(rows_v, old2_hbm.at[pl.ds(wid * BPW, BPW)])


@functools.partial(
    pl.kernel, mesh=_mesh,
    out_type=(),
    compiler_params=pltpu.CompilerParams(needs_layout_passes=False),
    scratch_types=[
        pltpu.VMEM_SHARED((BLK + SCRAP, D2), jnp.float32),
        pltpu.VMEM((NCH, 128), jnp.int32),
        pltpu.VMEM((NCH, 128), jnp.int32),
        pltpu.VMEM((NCH, 128), jnp.int32),
        pltpu.VMEM((NCH, 128), jnp.int32),
        pltpu.VMEM((128, D2), jnp.float32),
    ],
)
def _sc_scatter(bank_ref, delta2_hbm, idx_hbm,
                acc_sh, idx_v, selj_v, selg_v, sell_v, buf_v):
    cid = lax.axis_index("c")
    sid = lax.axis_index("s")
    pltpu.sync_copy(idx_hbm.at[sid], idx_v)

    def block_body(b):
        blk = cid * (NBLK // NC) + b
        start = jnp.minimum(blk * BLK, M2 - BLK)
        lo = blk * BLK
        hi = jnp.minimum(lo + BLK, M2)
        for r in range(NCH):
            for k in range(8):
                lane = lax.iota(jnp.int32, 16) + (r * 128 + k * 16)
                sl = pl.ds(k * 16, 16)
                selj_v[r, sl] = (lane + sid * 64) & (BPT - 1)
                selg_v[r, sl] = M2 + ((lane + sid * 64) & (HS - 1))
                sell_v[r, sl] = BLK + ((lane + sid * 32) & (SCRAP - 1))
        off = jnp.zeros((16,), jnp.int32)
        ones = jnp.ones((16,), jnp.int32)
        for i in range(BPT // 16):
            iv = idx_v[i // 8, pl.ds((i % 8) * 16, 16)]
            inb = (iv >= lo) & (iv < hi)
            inc = plsc.cumsum(ones, mask=inb)
            pos = off + inc - 1
            row = lax.shift_right_logical(pos, 7)
            col = pos & 127
            jrow = lax.iota(jnp.int32, 16) + (sid * BPT + i * 16)
            plsc.store_scatter(selj_v, [row, col], jrow, mask=inb)
            plsc.store_scatter(selg_v, [row, col], iv, mask=inb)
            plsc.store_scatter(sell_v, [row, col], iv - start, mask=inb)
            off = off + plsc.cummax(lax.rev(inc, (0,)))
        for c in range(NCH):
            @pl.when(jnp.any(off > c * 128))
            def _():
                pltpu.sync_copy(bank_ref.at[selg_v.at[c]], buf_v)
                pltpu.sync_copy(buf_v, acc_sh.at[sell_v.at[c]])
        plsc.subcore_barrier()
        for c in range(NCH):
            @pl.when(jnp.any(off > c * 128))
            def _():
                pltpu.sync_copy(delta2_hbm.at[selj_v.at[c]], buf_v)
                pltpu.sync_copy(buf_v, acc_sh.at[sell_v.at[c]], add=True)
        plsc.subcore_barrier()
        for c in range(NCH):
            @pl.when(jnp.any(off > c * 128))
            def _():
                pltpu.sync_copy(acc_sh.at[sell_v.at[c]], buf_v)
                pltpu.sync_copy(buf_v, bank_ref.at[selg_v.at[c]])
        plsc.subcore_barrier()

    for _b in range(NBLK // NC):
        block_body(jnp.int32(_b))


def _delta_body(old2_ref, val_ref, par_ref, wg_ref, bg_ref, wv_ref, out_ref):
    p = par_ref[0]
    old2 = old2_ref[...]
    old = old2[:, :D] * (1.0 - p) + old2[:, D:] * p
    v = val_ref[...]
    pre = jnp.dot(old + v, wg_ref[...], preferred_element_type=jnp.float32)
    gate = jax.nn.sigmoid(pre + bg_ref[...])
    upd = jnp.tanh(jnp.dot(v, wv_ref[...], preferred_element_type=jnp.float32))
    d = gate * (upd - old)
    out_ref[...] = jnp.concatenate([d * (1.0 - p), d * p], axis=1)


_TCB = 2048


def _tc_delta(old2, values, par, W_gate, b_gate, W_val):
    return pl.pallas_call(
        _delta_body,
        grid=(B // _TCB,),
        in_specs=[
            pl.BlockSpec((_TCB, D2), lambda i: (i, 0)),
            pl.BlockSpec((_TCB, D), lambda i: (i, 0)),
            pl.BlockSpec((1, _TCB, 1), lambda i: (i, 0, 0)),
            pl.BlockSpec((D, D), lambda i: (0, 0)),
            pl.BlockSpec((1, D), lambda i: (0, 0)),
            pl.BlockSpec((D, D), lambda i: (0, 0)),
        ],
        out_specs=pl.BlockSpec((_TCB, D2), lambda i: (i, 0)),
        out_shape=jax.ShapeDtypeStruct((B, D2), jnp.float32),
    )(old2, values, par, W_gate, b_gate, W_val)


def kernel(memory, indices, values, W_gate, b_gate, W_val):
    idx = indices.astype(jnp.int32)
    idxp = idx >> 1
    par = (idx & 1).astype(jnp.float32).reshape(B // _TCB, _TCB, 1)
    mem2p = jnp.concatenate(
        [memory.reshape(M2, D2), jnp.zeros((HS, D2), jnp.float32)])
    bank = jax.new_ref(mem2p)
    old2 = _sc_gather(bank, idxp.reshape(NW, BPW // 128, 128))
    delta2 = _tc_delta(old2, values, par, W_gate, b_gate.reshape(1, D), W_val)
    _sc_scatter(bank, delta2, idxp.reshape(NS, BPT // 128, 128))
    return bank[...][:M2].reshape(M, D)
